# Initial kernel scaffold; baseline (speedup 1.0000x reference)
import jax
import jax.numpy as jnp
from jax.experimental import pallas as pl


def kernel(x, edge_index, routing_factor, W_msg, W1, b1, W2, b2, gamma1, beta1, gamma2, beta2):
    raise NotImplementedError("write your pallas kernel here")


# speedup vs baseline: 4.1592x; 4.1592x over previous
import functools

import jax
import jax.numpy as jnp
from jax import lax
from jax.experimental import pallas as pl
from jax.experimental.pallas import tpu as pltpu
from jax.experimental.pallas import tpu_sc as plsc

N = 10000
D = 128
E = 320000
NUM_CORES = 2
NUM_SUBCORES = 16
NW = NUM_CORES * NUM_SUBCORES
EDGES_PER_W = E // NW
CHUNK = 80
NCHUNK = EDGES_PER_W // CHUNK
NPAD = 10240
ROWS_PER_TILE = NPAD // NUM_SUBCORES
ZROWS = 128
LANES = 16


def _sc_scatter_partials(src, dst, rf, x2):
    mesh = plsc.VectorSubcoreMesh(
        core_axis_name="c", subcore_axis_name="s",
        num_cores=NUM_CORES, num_subcores=NUM_SUBCORES)

    @functools.partial(
        pl.kernel,
        out_type=jax.ShapeDtypeStruct((NUM_CORES, NPAD, D), jnp.float32),
        mesh=mesh,
        scratch_types=[
            pltpu.VMEM_SHARED((NPAD, D), jnp.float32),
            pltpu.VMEM((CHUNK,), jnp.int32),
            pltpu.VMEM((CHUNK,), jnp.int32),
            pltpu.VMEM((CHUNK + LANES,), jnp.float32),
            pltpu.VMEM((CHUNK, D), jnp.float32),
            pltpu.VMEM((ZROWS, D), jnp.float32),
            pltpu.SemaphoreType.DMA,
        ],
    )
    def k(src_hbm, dst_hbm, rf_hbm, x_hbm, out_hbm,
          acc_sh, srcv, dstv, rfv, rows, zbuf, sem):
        cid = lax.axis_index("c")
        sid = lax.axis_index("s")
        wid = cid * NUM_SUBCORES + sid

        zeros16 = jnp.zeros((LANES,), jnp.float32)

        def zrow(i, carry):
            for j in range(D // LANES):
                zbuf[i, pl.ds(j * LANES, LANES)] = zeros16
            return carry

        lax.fori_loop(0, ZROWS, zrow, 0)
        base_row = sid * ROWS_PER_TILE
        for t in range(ROWS_PER_TILE // ZROWS):
            pltpu.sync_copy(zbuf, acc_sh.at[pl.ds(base_row + t * ZROWS, ZROWS)])
        plsc.subcore_barrier()

        ebase = wid * EDGES_PER_W

        def chunk_body(c, carry):
            off = ebase + c * CHUNK
            pltpu.sync_copy(src_hbm.at[pl.ds(off, CHUNK)], srcv)
            pltpu.sync_copy(dst_hbm.at[pl.ds(off, CHUNK)], dstv)
            pltpu.sync_copy(rf_hbm.at[pl.ds(off, CHUNK)], rfv.at[pl.ds(0, CHUNK)])
            pltpu.async_copy(x_hbm.at[srcv], rows, sem).wait()

            def srow(i, c2):
                w = rfv[pl.ds(i, LANES)][0]
                for j in range(D // LANES):
                    sl = pl.ds(j * LANES, LANES)
                    rows[i, sl] = rows[i, sl] * w
                return c2

            lax.fori_loop(0, CHUNK, srow, 0)
            pltpu.sync_copy(rows, acc_sh.at[dstv], add=True)
            return carry

        lax.fori_loop(0, NCHUNK, chunk_body, 0)
        plsc.subcore_barrier()

        pltpu.sync_copy(acc_sh.at[pl.ds(base_row, ROWS_PER_TILE)],
                        out_hbm.at[cid, pl.ds(base_row, ROWS_PER_TILE)])

    return k(src, dst, rf, x2)


def _gelu(v):
    return 0.5 * v * (1.0 + lax.erf(v * 0.7071067811865476))


def _layer_norm(v, g, b):
    mu = jnp.mean(v, axis=-1, keepdims=True)
    var = jnp.mean(jnp.square(v - mu), axis=-1, keepdims=True)
    return (v - mu) * lax.rsqrt(var + 1e-5) * g + b


def _dense_body(x_ref, ap_ref, wm_ref, w1_ref, b1_ref, w2_ref, b2_ref,
                g1_ref, be1_ref, g2_ref, be2_ref, o_ref):
    cdims = (((1,), (1,)), ((), ()))
    a = ap_ref[0] + ap_ref[1]
    agg = lax.dot_general(a, wm_ref[...], cdims, preferred_element_type=jnp.float32)
    t = x_ref[...] + _gelu(agg)
    u = _layer_norm(t, g1_ref[...], be1_ref[...])
    h = _gelu(lax.dot_general(u, w1_ref[...], cdims,
                              preferred_element_type=jnp.float32) + b1_ref[...])
    h2 = lax.dot_general(h, w2_ref[...], cdims,
                         preferred_element_type=jnp.float32) + b2_ref[...]
    o_ref[...] = _layer_norm(u + h2, g2_ref[...], be2_ref[...])


def _tc_dense(x2, partials, W_msg, W1, b1, W2, b2, g1, be1, g2, be2, interpret=False):
    R = 1000
    grid = (N // R,)
    row_spec = pl.BlockSpec((R, D), lambda i: (i, 0))

    def whole(shape):
        return pl.BlockSpec(shape, lambda i: tuple(0 for _ in shape))

    return pl.pallas_call(
        _dense_body,
        grid=grid,
        in_specs=[
            row_spec,
            pl.BlockSpec((2, R, D), lambda i: (0, i, 0)),
            whole((D, D)), whole((2 * D, D)), whole((1, 2 * D)),
            whole((D, 2 * D)), whole((1, D)),
            whole((1, D)), whole((1, D)), whole((1, D)), whole((1, D)),
        ],
        out_specs=row_spec,
        out_shape=jax.ShapeDtypeStruct((N, D), jnp.float32),
        interpret=interpret,
    )(x2, partials, W_msg, W1, b1.reshape(1, -1), W2,
      b2.reshape(1, -1), g1.reshape(1, -1), be1.reshape(1, -1),
      g2.reshape(1, -1), be2.reshape(1, -1))


def kernel(x, edge_index, routing_factor, W_msg, W1, b1, W2, b2,
           gamma1, beta1, gamma2, beta2):
    x2 = x[0]
    partials = _sc_scatter_partials(edge_index[0], edge_index[1],
                                    routing_factor, x2)
    out = _tc_dense(x2, partials, W_msg, W1, b1, W2, b2,
                    gamma1, beta1, gamma2, beta2)
    return out[None]
